# Initial kernel scaffold; baseline (speedup 1.0000x reference)
import jax
import jax.numpy as jnp
from jax.experimental import pallas as pl


def kernel(x, geom_xy):
    raise NotImplementedError("write your pallas kernel here")


# speedup vs baseline: 2.3030x; 2.3030x over previous
import functools

import jax
import jax.numpy as jnp
from jax import lax
from jax.experimental import pallas as pl
from jax.experimental.pallas import tpu as pltpu
from jax.experimental.pallas import tpu_sc as plsc

NX = 128
NY = 128
C = 80
NSEG = NX * NY
NPTS = 249216
CHUNK = 384
NCHUNKS = NPTS // CHUNK
NW = 32
ROWS_PER_TILE = NSEG // 16


_mesh = plsc.VectorSubcoreMesh(core_axis_name="c", subcore_axis_name="s")


@functools.partial(
    pl.kernel,
    out_type=jax.ShapeDtypeStruct((2, NSEG, C), jnp.float32),
    mesh=_mesh,
    compiler_params=pltpu.CompilerParams(use_tc_tiling_on_sc=False),
    scratch_types=[
        pltpu.VMEM_SHARED((NSEG, C), jnp.float32),
        pltpu.VMEM((CHUNK, C), jnp.float32),
        pltpu.VMEM((CHUNK,), jnp.int32),
        pltpu.VMEM((CHUNK,), jnp.int32),
        pltpu.VMEM((CHUNK // 128, 128), jnp.int32),
        pltpu.VMEM((16, C), jnp.float32),
    ],
)
def _sc_bev_scatter(x_hbm, gx_hbm, gy_hbm, out_hbm, accum, xbuf, gxbuf, gybuf,
                    ranks, zbuf):
    cid_core = lax.axis_index("c")
    sid = lax.axis_index("s")
    wid = sid * 2 + cid_core

    zeros16 = jnp.zeros((16,), jnp.float32)

    def _zrow(i, _):
        for k in range(C // 16):
            zbuf[i, pl.ds(16 * k, 16)] = zeros16
        return 0

    lax.fori_loop(0, 16, _zrow, 0)
    row0 = sid * ROWS_PER_TILE

    def _zdma(j, _):
        pltpu.sync_copy(zbuf, accum.at[pl.ds(row0 + 16 * j, 16), :])
        return 0

    lax.fori_loop(0, ROWS_PER_TILE // 16, _zdma, 0)
    plsc.subcore_barrier()

    n_extra = NCHUNKS - NW * (NCHUNKS // NW)
    nj = jnp.where(wid < n_extra, NCHUNKS // NW + 1, NCHUNKS // NW)

    def _chunk(j, _):
        base = (wid + NW * j) * CHUNK
        pltpu.sync_copy(x_hbm.at[pl.ds(base, CHUNK), :], xbuf)
        pltpu.sync_copy(gx_hbm.at[pl.ds(base, CHUNK)], gxbuf)
        pltpu.sync_copy(gy_hbm.at[pl.ds(base, CHUNK)], gybuf)
        for sb in range(CHUNK // 128):
            for i in range(8):
                off = sb * 128 + i * 16
                r = gxbuf[pl.ds(off, 16)] * NY + gybuf[pl.ds(off, 16)]
                ranks[sb, pl.ds(i * 16, 16)] = r
        for sb in range(CHUNK // 128):
            pltpu.sync_copy(
                xbuf.at[pl.ds(sb * 128, 128), :],
                accum.at[ranks.at[sb]],
                add=True,
            )
        return 0

    lax.fori_loop(0, nj, _chunk, 0)
    plsc.subcore_barrier()

    pltpu.sync_copy(
        accum.at[pl.ds(row0, ROWS_PER_TILE), :],
        out_hbm.at[cid_core, pl.ds(row0, ROWS_PER_TILE), :],
    )


def _tc_combine_body(p_ref, o_ref):
    s = p_ref[0] + p_ref[1]
    o_ref[...] = s.T


_TC_BLK = 1024


def _tc_combine(partials):
    return pl.pallas_call(
        _tc_combine_body,
        grid=(NSEG // _TC_BLK,),
        in_specs=[pl.BlockSpec((2, _TC_BLK, C), lambda j: (0, j, 0))],
        out_specs=pl.BlockSpec((C, _TC_BLK), lambda j: (0, j)),
        out_shape=jax.ShapeDtypeStruct((C, NSEG), jnp.float32),
    )(partials)


@jax.jit
def kernel(x, geom_xy):
    gx = geom_xy[:, 0]
    gy = geom_xy[:, 1]
    partials = _sc_bev_scatter(x, gx, gy)
    out = _tc_combine(partials)
    return out.reshape(1, C, NX, NY)
